# Initial kernel scaffold; baseline (speedup 1.0000x reference)
import jax
import jax.numpy as jnp
from jax.experimental import pallas as pl


def kernel(x, edge_index, W1, b1, W2, b2):
    raise NotImplementedError("write your pallas kernel here")


# speedup vs baseline: 20.9751x; 20.9751x over previous
import functools
import jax
import jax.numpy as jnp
from jax import lax
from jax.experimental import pallas as pl
from jax.experimental.pallas import tpu as pltpu
from jax.experimental.pallas import tpu_sc as plsc

NN = 10000
NE = 320000
DIN = 128
DHID = 128
NCLS = 40

NC = 2
NS = 16
NW = NC * NS
EPW = NE // NW
CH = 80
NCHUNK = EPW // CH
NNP = 10240
RPT = NNP // NS

_mesh = plsc.VectorSubcoreMesh(core_axis_name="c", subcore_axis_name="s")


def _zero_shared(zb, shared, s, d):
    nr = zb.shape[0]
    def zrow(i, _):
        for k in range(d // 16):
            zb[i, pl.ds(k * 16, 16)] = jnp.zeros((16,), jnp.float32)
        return _
    lax.fori_loop(0, nr, zrow, 0)
    for r in range(RPT // nr):
        pltpu.sync_copy(zb, shared.at[pl.ds(s * RPT + r * nr, nr)])


def _copy_out(shared, out_hbm, c, s):
    pltpu.sync_copy(shared.at[pl.ds(s * RPT, RPT)],
                    out_hbm.at[c, pl.ds(s * RPT, RPT)])


@functools.partial(
    pl.kernel,
    out_type=jax.ShapeDtypeStruct((NW, NN), jnp.float32),
    mesh=_mesh,
    scratch_types=[
        pltpu.VMEM((EPW,), jnp.int32),
        pltpu.VMEM((NN,), jnp.float32),
    ],
    compiler_params=pltpu.CompilerParams(needs_layout_passes=False),
)
def _sc_deg(dst_hbm, out_hbm, idxv, degv):
    c = lax.axis_index("c")
    s = lax.axis_index("s")
    wid = s * NC + c
    def zr(i, _):
        degv[pl.ds(i * 16, 16)] = jnp.zeros((16,), jnp.float32)
        return _
    lax.fori_loop(0, NN // 16, zr, 0)
    pltpu.sync_copy(dst_hbm.at[wid], idxv)
    ones = jnp.ones((16,), jnp.float32)
    def step(i, _):
        plsc.addupdate_scatter(degv, [idxv[pl.ds(i * 16, 16)]], ones)
        return _
    lax.fori_loop(0, EPW // 16, step, 0)
    pltpu.sync_copy(degv, out_hbm.at[wid])


def _make_sc_scatter(d):
    @functools.partial(
        pl.kernel,
        out_type=jax.ShapeDtypeStruct((NC, NNP, d), jnp.float32),
        mesh=_mesh,
        scratch_types=[
            pltpu.VMEM((NCHUNK, CH), jnp.int32),
            pltpu.VMEM((NCHUNK, CH), jnp.int32),
            pltpu.VMEM((CH, d), jnp.float32),
            pltpu.VMEM_SHARED((NNP, d), jnp.float32),
            pltpu.SemaphoreType.DMA,
        ],
    )
    def _sc_scatter(src_hbm, dst_hbm, h_hbm, out_hbm,
                    srcv, dstv, rows, shared, sem):
        c = lax.axis_index("c")
        s = lax.axis_index("s")
        wid = s * NC + c
        _zero_shared(rows, shared, s, d)
        pltpu.sync_copy(src_hbm.at[wid], srcv)
        pltpu.sync_copy(dst_hbm.at[wid], dstv)
        plsc.subcore_barrier()
        def chunk(j, _):
            cp = pltpu.make_async_copy(h_hbm.at[srcv.at[j]], rows, sem)
            cp.start()
            cp.wait()
            pltpu.sync_copy(rows, shared.at[dstv.at[j]], add=True)
            return _
        lax.fori_loop(0, NCHUNK, chunk, 0)
        plsc.subcore_barrier()
        _copy_out(shared, out_hbm, c, s)
    return _sc_scatter

_sc_scatter128 = _make_sc_scatter(DHID)


def _dinv_of(dp):
    return lax.rsqrt(1.0 + jnp.sum(dp, axis=0))[:, None]


def _tc_a_body(x_ref, w1_ref, dp_ref, hp_ref):
    dinv = _dinv_of(dp_ref[...])
    h = jnp.dot(x_ref[...], w1_ref[...], preferred_element_type=jnp.float32)
    hp_ref[...] = h * dinv


def _tc_c_body(p_ref, hp_ref, dp_ref, b1_ref, g_ref):
    dinv = _dinv_of(dp_ref[...])
    ssum = p_ref[0, :NN] + p_ref[1, :NN] + hp_ref[...]
    h1 = jnp.maximum(ssum * dinv + b1_ref[...], 0.0)
    g_ref[...] = h1 * dinv


def _tc_e_body(q_ref, g_ref, dp_ref, w2_ref, b2_ref, out_ref):
    dinv = _dinv_of(dp_ref[...])
    t = q_ref[0, :NN] + q_ref[1, :NN] + g_ref[...]
    z = jnp.dot(t, w2_ref[...], preferred_element_type=jnp.float32) * dinv
    z = z + b2_ref[...]
    m = jnp.max(z, axis=1, keepdims=True)
    lse = jnp.log(jnp.sum(jnp.exp(z - m), axis=1, keepdims=True))
    out_ref[...] = z - m - lse


def _tc_a(x, w1, dp):
    return pl.pallas_call(
        _tc_a_body,
        out_shape=jax.ShapeDtypeStruct((NN, DHID), jnp.float32),
    )(x, w1, dp)


def _tc_c(p, hp, dp, b1):
    return pl.pallas_call(
        _tc_c_body,
        out_shape=jax.ShapeDtypeStruct((NN, DHID), jnp.float32),
    )(p, hp, dp, b1)


def _tc_e(q, g, dp, w2, b2):
    return pl.pallas_call(
        _tc_e_body,
        out_shape=jax.ShapeDtypeStruct((NN, NCLS), jnp.float32),
    )(q, g, dp, w2, b2)


@jax.jit
def kernel(x, edge_index, W1, b1, W2, b2):
    src = edge_index[0].reshape(NW, NCHUNK, CH)
    dst = edge_index[1].reshape(NW, NCHUNK, CH)
    dp = _sc_deg(edge_index[1].reshape(NW, EPW))
    hp = _tc_a(x, W1, dp)
    p = _sc_scatter128(src, dst, hp)
    g = _tc_c(p, hp, dp, b1.reshape(1, DHID))
    q = _sc_scatter128(src, dst, g)
    return _tc_e(q, g, dp, W2, b2.reshape(1, NCLS))
